# baseline (speedup 1.0000x reference)
import functools

import jax
import jax.numpy as jnp
from jax import lax
from jax.experimental import pallas as pl
from jax.experimental.pallas import tpu as pltpu
from jax.experimental.pallas import tpu_sc as plsc

_TIME = 288
_B, _T, _N, _C = 64, 12, 8192, 3
_F = 64
_L = 16
_NC, _NS = 2, 16
_NW = _NC * _NS
_NPW = _N // _NW
_NVEC = _NPW // _L
_WPAD = 8
_CT = _TIME * _WPAD
_QF = 16
_NQ = _F // _QF


def _tec_body(day_hbm, week_hbm, dayt_hbm, weekt_hbm, out_hbm,
              stage_v, cidx_v, dayt_v, weekt_v, ctab_v, outbuf_v,
              sem0, sem1):
    cid = lax.axis_index("c")
    sid = lax.axis_index("s")
    wid = sid * _NC + cid
    n0 = wid * _NPW

    pltpu.sync_copy(dayt_hbm, dayt_v)
    pltpu.sync_copy(weekt_hbm, weekt_v)

    pltpu.sync_copy(day_hbm.at[:, pl.ds(n0, _NPW)], stage_v)

    def day_idx_body(i, _):
        b = i // _NVEC
        j = i - b * _NVEC
        v = stage_v[b, pl.ds(j * _L, _L)]
        d = lax.convert_element_type(v * float(_TIME), jnp.int32)
        cidx_v[b, pl.ds(j * _L, _L)] = jnp.clip(d, 0, _TIME - 1) * _WPAD
        return 0

    lax.fori_loop(0, _B * _NVEC, day_idx_body, 0)

    pltpu.sync_copy(week_hbm.at[:, pl.ds(n0, _NPW)], stage_v)

    def week_idx_body(i, _):
        b = i // _NVEC
        j = i - b * _NVEC
        v = stage_v[b, pl.ds(j * _L, _L)]
        w = lax.convert_element_type(v, jnp.int32)
        sl = (b, pl.ds(j * _L, _L))
        cidx_v[sl] = cidx_v[sl] + jnp.clip(w, 0, 6)
        return 0

    lax.fori_loop(0, _B * _NVEC, week_idx_body, 0)

    iota = lax.broadcasted_iota(jnp.int32, (_L,), 0)
    wsel = jnp.bitwise_and(iota, _WPAD - 1)
    dsel = lax.shift_right_logical(iota, 3)

    sems = (sem0, sem1)

    for q in range(_NQ):
        f0 = q * _QF

        for fi in range(_QF):
            f = f0 + fi
            wrow = plsc.load_gather(weekt_v, [f * _WPAD + wsel])

            def build_body(j, _, f=f, fi=fi, wrow=wrow):
                dvals = plsc.load_gather(dayt_v, [f * _TIME + j * 2 + dsel])
                ctab_v[pl.ds(fi * _CT + j * _L, _L)] = dvals + wrow
                return 0

            lax.fori_loop(0, _CT // _L, build_body, 0)

        def batch_pair(bb, _, q=q, f0=f0):
            for k in range(2):
                b = bb * 2 + k

                @pl.when(bb > 0)
                def _wait(k=k, b=b):
                    pltpu.make_async_copy(
                        outbuf_v.at[k],
                        out_hbm.at[b, pl.ds(f0, _QF), pl.ds(n0, _NPW)],
                        sems[k],
                    ).wait()

                def vec_body(j, _, k=k, b=b):
                    cvec = cidx_v[b, pl.ds(j * _L, _L)]
                    for fi in range(_QF):
                        g = plsc.load_gather(ctab_v, [cvec + fi * _CT])
                        outbuf_v[k, fi, pl.ds(j * _L, _L)] = g
                    return 0

                lax.fori_loop(0, _NVEC, vec_body, 0)
                pltpu.async_copy(
                    outbuf_v.at[k],
                    out_hbm.at[b, pl.ds(f0, _QF), pl.ds(n0, _NPW)],
                    sems[k],
                )
            return 0

        lax.fori_loop(0, _B // 2, batch_pair, 0)

        for k in range(2):
            pltpu.make_async_copy(
                outbuf_v.at[k],
                out_hbm.at[_B - 2 + k, pl.ds(f0, _QF), pl.ds(n0, _NPW)],
                sems[k],
            ).wait()


@functools.partial(
    pl.kernel,
    mesh=plsc.VectorSubcoreMesh(core_axis_name="c", subcore_axis_name="s"),
    out_type=jax.ShapeDtypeStruct((_B, _F, _N), jnp.float32),
    compiler_params=pltpu.CompilerParams(needs_layout_passes=False),
    scratch_types=[
        pltpu.VMEM((_B, _NPW), jnp.float32),
        pltpu.VMEM((_B, _NPW), jnp.int32),
        pltpu.VMEM((_F * _TIME,), jnp.float32),
        pltpu.VMEM((_F * _WPAD,), jnp.float32),
        pltpu.VMEM((_QF * _CT,), jnp.float32),
        pltpu.VMEM((2, _QF, _NPW), jnp.float32),
        pltpu.SemaphoreType.DMA,
        pltpu.SemaphoreType.DMA,
    ],
)
def _sc_lookup(day_hbm, week_hbm, dayt_hbm, weekt_hbm, out_hbm,
               stage_v, cidx_v, dayt_v, weekt_v, ctab_v, outbuf_v,
               sem0, sem1):
    _tec_body(day_hbm, week_hbm, dayt_hbm, weekt_hbm, out_hbm,
              stage_v, cidx_v, dayt_v, weekt_v, ctab_v, outbuf_v,
              sem0, sem1)


def kernel(x, time_day, time_week):
    day_frac = x[:, _T - 1, :, 1]
    week_val = x[:, _T - 1, :, 2]
    dayt = jnp.transpose(time_day).reshape(-1)
    weekt = jnp.concatenate(
        [jnp.transpose(time_week),
         jnp.zeros((_F, _WPAD - 7), jnp.float32)], axis=1).reshape(-1)
    out = _sc_lookup(day_frac, week_val, dayt, weekt)
    return out[..., None]
